# baseline (speedup 1.0000x reference)
import functools

import jax
import jax.numpy as jnp
from jax import lax
from jax.experimental import pallas as pl
from jax.experimental.pallas import tpu as pltpu
from jax.experimental.pallas import tpu_sc as plsc

_N = 10000
_E = 320000
_IN = 128
_HID = 256
_OUT = 128

_NP = 10240
_GRP = 128
_NW = 32
_G = 80
_EPW = _GRP * _G
_EPAD = _EPW * _NW
_RPT = _NP // 16
_NBUF = 2
_GC0 = 96
_GC1 = 64

_mesh = plsc.VectorSubcoreMesh(core_axis_name="c", subcore_axis_name="s")
_sc_params = pltpu.CompilerParams(needs_layout_passes=False)


@functools.partial(
    pl.kernel,
    out_type=[
        jax.ShapeDtypeStruct((_NW, _NP), jnp.float32),
        jax.ShapeDtypeStruct((_NW, _NP), jnp.float32),
    ],
    mesh=_mesh,
    compiler_params=_sc_params,
    scratch_types=[
        pltpu.VMEM((_EPW,), jnp.int32),
        pltpu.VMEM((_EPW,), jnp.int32),
        pltpu.VMEM((_NP,), jnp.float32),
        pltpu.VMEM((_NP,), jnp.float32),
    ],
)
def _sc_degrees(src_hbm, dst_hbm, outs_hbm, outd_hbm, sidx_v, didx_v, cs_v, cd_v):
    c = lax.axis_index("c")
    s = lax.axis_index("s")
    w = s * 2 + c
    zero16 = jnp.zeros((16,), jnp.float32)
    ones16 = jnp.ones((16,), jnp.float32)

    def zb(i, carry):
        cs_v[pl.ds(i * 16, 16)] = zero16
        cd_v[pl.ds(i * 16, 16)] = zero16
        return carry

    lax.fori_loop(0, _NP // 16, zb, 0)

    pltpu.sync_copy(src_hbm.at[pl.ds(w * _EPW, _EPW)], sidx_v)
    pltpu.sync_copy(dst_hbm.at[pl.ds(w * _EPW, _EPW)], didx_v)

    def cnt(i, carry):
        plsc.addupdate_scatter(cs_v, [sidx_v[pl.ds(i * 16, 16)]], ones16)
        plsc.addupdate_scatter(cd_v, [didx_v[pl.ds(i * 16, 16)]], ones16)
        return carry

    lax.fori_loop(0, _EPW // 16, cnt, 0)

    pltpu.sync_copy(cs_v, outs_hbm.at[w])
    pltpu.sync_copy(cd_v, outd_hbm.at[w])


@functools.partial(
    pl.kernel,
    out_type=jax.ShapeDtypeStruct((2, _NP, 128), jnp.float32),
    mesh=_mesh,
    compiler_params=_sc_params,
    scratch_types=[
        pltpu.VMEM((_GRP,), jnp.int32),
        pltpu.VMEM((_GRP,), jnp.int32),
        pltpu.VMEM((_GRP, 128), jnp.float32),
        pltpu.VMEM_SHARED((_NP, 128), jnp.float32),
        pltpu.SemaphoreType.DMA,
    ],
)
def _sc_aggregate(xs_hbm, src_hbm, dst_hbm, out_hbm, sidx_v, didx_v, rows_v,
                  acc_sh, sem):
    c = lax.axis_index("c")
    s = lax.axis_index("s")
    zero16 = jnp.zeros((16,), jnp.float32)

    def zr(i, carry):
        r = i // 8
        j = i - r * 8
        rows_v[r, pl.ds(j * 16, 16)] = zero16
        return carry

    lax.fori_loop(0, _GRP * 8, zr, 0)
    for k in range(_RPT // _GRP):
        pltpu.sync_copy(rows_v, acc_sh.at[pl.ds(s * _RPT + k * _GRP, _GRP)])
    plsc.subcore_barrier()

    if _GC0 == _GC1:
        ng = _GC0
        base = (s * 2 + c) * _GC0 * _GRP
    else:
        ng = jnp.where(c == 0, _GC0, _GC1)
        base = jnp.where(c == 0, s * _GC0, 16 * _GC0 + s * _GC1) * _GRP

    def body(g, carry):
        off = base + g * _GRP
        pltpu.sync_copy(src_hbm.at[pl.ds(off, _GRP)], sidx_v)
        pltpu.sync_copy(dst_hbm.at[pl.ds(off, _GRP)], didx_v)
        pltpu.async_copy(xs_hbm.at[sidx_v], rows_v, sem).wait()
        pltpu.sync_copy(rows_v, acc_sh.at[didx_v], add=True)
        return carry

    lax.fori_loop(0, ng, body, 0)
    plsc.subcore_barrier()
    pltpu.sync_copy(acc_sh.at[pl.ds(s * _RPT, _RPT)],
                    out_hbm.at[c, pl.ds(s * _RPT, _RPT)])


_RB = 2048


def _scale_body(x_ref, d_ref, o_ref):
    o_ref[...] = x_ref[...] * d_ref[...]


def _layer_body(p_ref, din_ref, dout_ref, w1_ref, b1_ref, w2_ref, o_ref):
    t = (p_ref[0] + p_ref[1]) * din_ref[...]
    u = jnp.dot(t, w1_ref[...], preferred_element_type=jnp.float32) + b1_ref[0:1, :]
    h = jnp.maximum(u, 0.0)
    o_ref[...] = jnp.dot(h, w2_ref[...], preferred_element_type=jnp.float32) * dout_ref[...]


def _final_body(p_ref, din_ref, b2_ref, o_ref):
    o_ref[...] = (p_ref[0] + p_ref[1]) * din_ref[...] + b2_ref[0:1, :]


def _tc_scale(x, d):
    return pl.pallas_call(
        _scale_body,
        grid=(_NP // _RB,),
        in_specs=[
            pl.BlockSpec((_RB, 128), lambda i: (i, 0)),
            pl.BlockSpec((_RB, 128), lambda i: (i, 0)),
        ],
        out_specs=pl.BlockSpec((_RB, 128), lambda i: (i, 0)),
        out_shape=jax.ShapeDtypeStruct((_NP, 128), jnp.float32),
    )(x, d)


def _tc_layer(parts, din, dout, W1, b1b, W2):
    return pl.pallas_call(
        _layer_body,
        grid=(_NP // _RB,),
        in_specs=[
            pl.BlockSpec((2, _RB, 128), lambda i: (0, i, 0)),
            pl.BlockSpec((_RB, 128), lambda i: (i, 0)),
            pl.BlockSpec((_RB, 128), lambda i: (i, 0)),
            pl.BlockSpec((_IN, _HID), lambda i: (0, 0)),
            pl.BlockSpec((8, _HID), lambda i: (0, 0)),
            pl.BlockSpec((_HID, _OUT), lambda i: (0, 0)),
        ],
        out_specs=pl.BlockSpec((_RB, 128), lambda i: (i, 0)),
        out_shape=jax.ShapeDtypeStruct((_NP, 128), jnp.float32),
    )(parts, din, dout, W1, b1b, W2)


def _tc_final(parts, din, b2b):
    return pl.pallas_call(
        _final_body,
        grid=(_NP // _RB,),
        in_specs=[
            pl.BlockSpec((2, _RB, 128), lambda i: (0, i, 0)),
            pl.BlockSpec((_RB, 128), lambda i: (i, 0)),
            pl.BlockSpec((8, 128), lambda i: (0, 0)),
        ],
        out_specs=pl.BlockSpec((_RB, 128), lambda i: (i, 0)),
        out_shape=jax.ShapeDtypeStruct((_NP, 128), jnp.float32),
    )(parts, din, b2b)


def kernel(features, edge_index, W1, b1, W2, b2):
    src = edge_index[0]
    dst = edge_index[1]
    pad = _EPAD - _E
    pad_idx = _N + (jnp.arange(pad, dtype=jnp.int32) % (_NP - _N))
    src_p = jnp.concatenate([src, pad_idx])
    dst_p = jnp.concatenate([dst, pad_idx])
    feat_p = jnp.pad(features, ((0, _NP - _N), (0, 0)))

    degs, degd = _sc_degrees(src_p, dst_p)
    deg_out = jnp.sum(degs, axis=0)
    deg_in = jnp.sum(degd, axis=0)
    d_out = lax.rsqrt(jnp.clip(deg_out, 1.0))
    d_in = lax.rsqrt(jnp.clip(deg_in, 1.0))
    d_out_b = jnp.broadcast_to(d_out[:, None], (_NP, 128))
    d_in_b = jnp.broadcast_to(d_in[:, None], (_NP, 128))
    b1b = jnp.broadcast_to(b1[None, :], (8, _HID))
    b2b = jnp.broadcast_to(b2[None, :], (8, _OUT))

    xs = _tc_scale(feat_p, d_out_b)
    parts1 = _sc_aggregate(xs, src_p, dst_p)
    z = _tc_layer(parts1, d_in_b, d_out_b, W1, b1b, W2)
    parts2 = _sc_aggregate(z, src_p, dst_p)
    out_full = _tc_final(parts2, d_in_b, b2b)
    return out_full[:_N]


# speedup vs baseline: 2.0016x; 2.0016x over previous
import functools

import jax
import jax.numpy as jnp
from jax import lax
from jax.experimental import pallas as pl
from jax.experimental.pallas import tpu as pltpu
from jax.experimental.pallas import tpu_sc as plsc

_N = 10000
_E = 320000
_IN = 128
_HID = 256
_OUT = 128

_NP = 10240
_GRP = 128
_NW = 32
_G = 80
_EPW = _GRP * _G
_EPAD = _EPW * _NW
_RPT = _NP // 16
_NBUF = 2
_GC0 = 80
_GC1 = 80

_mesh = plsc.VectorSubcoreMesh(core_axis_name="c", subcore_axis_name="s")
_sc_params = pltpu.CompilerParams(needs_layout_passes=False)


@functools.partial(
    pl.kernel,
    out_type=[
        jax.ShapeDtypeStruct((_NW, _NP), jnp.float32),
        jax.ShapeDtypeStruct((_NW, _NP), jnp.float32),
    ],
    mesh=_mesh,
    compiler_params=_sc_params,
    scratch_types=[
        pltpu.VMEM((_EPW,), jnp.int32),
        pltpu.VMEM((_EPW,), jnp.int32),
        pltpu.VMEM((_NP,), jnp.float32),
        pltpu.VMEM((_NP,), jnp.float32),
    ],
)
def _sc_degrees(src_hbm, dst_hbm, outs_hbm, outd_hbm, sidx_v, didx_v, cs_v, cd_v):
    c = lax.axis_index("c")
    s = lax.axis_index("s")
    w = s * 2 + c
    zero16 = jnp.zeros((16,), jnp.float32)
    ones16 = jnp.ones((16,), jnp.float32)

    def zb(i, carry):
        cs_v[pl.ds(i * 16, 16)] = zero16
        cd_v[pl.ds(i * 16, 16)] = zero16
        return carry

    lax.fori_loop(0, _NP // 16, zb, 0)

    pltpu.sync_copy(src_hbm.at[pl.ds(w * _EPW, _EPW)], sidx_v)
    pltpu.sync_copy(dst_hbm.at[pl.ds(w * _EPW, _EPW)], didx_v)

    def cnt(i, carry):
        plsc.addupdate_scatter(cs_v, [sidx_v[pl.ds(i * 16, 16)]], ones16)
        plsc.addupdate_scatter(cd_v, [didx_v[pl.ds(i * 16, 16)]], ones16)
        return carry

    lax.fori_loop(0, _EPW // 16, cnt, 0)

    pltpu.sync_copy(cs_v, outs_hbm.at[w])
    pltpu.sync_copy(cd_v, outd_hbm.at[w])


@functools.partial(
    pl.kernel,
    out_type=jax.ShapeDtypeStruct((2, _NP, 128), jnp.float32),
    mesh=_mesh,
    compiler_params=_sc_params,
    scratch_types=[
        pltpu.VMEM((2, _GRP), jnp.int32),
        pltpu.VMEM((2, _GRP), jnp.int32),
        pltpu.VMEM((_GRP, 128), jnp.float32),
        pltpu.VMEM((_GRP, 128), jnp.float32),
        pltpu.VMEM_SHARED((_NP, 128), jnp.float32),
        pltpu.SemaphoreType.DMA,
        pltpu.SemaphoreType.DMA,
    ],
)
def _sc_aggregate(xs_hbm, idx2_hbm, out_hbm, i0, i1, r0, r1, acc_sh, gs0, gs1):
    c = lax.axis_index("c")
    s = lax.axis_index("s")
    ibufs = (i0, i1)
    bufs = (r0, r1)
    gsems = (gs0, gs1)
    zero16 = jnp.zeros((16,), jnp.float32)
    base = (s * 2 + c) * _G

    def zr(i, carry):
        r = i // 8
        j = i - r * 8
        r0[r, pl.ds(j * 16, 16)] = zero16
        return carry

    lax.fori_loop(0, _GRP * 8, zr, 0)
    for k in range(_RPT // _GRP):
        pltpu.sync_copy(r0, acc_sh.at[pl.ds(s * _RPT + k * _GRP, _GRP)])

    for b in range(_NBUF):
        pltpu.sync_copy(idx2_hbm.at[base + b], ibufs[b])
        pltpu.async_copy(xs_hbm.at[ibufs[b].at[0]], bufs[b], gsems[b])
    plsc.subcore_barrier()

    def round_body(r, carry):
        for b in range(_NBUF):
            g = r * _NBUF + b
            pltpu.make_async_copy(xs_hbm.at[ibufs[b].at[0]], bufs[b],
                                  gsems[b]).wait()
            pltpu.sync_copy(bufs[b], acc_sh.at[ibufs[b].at[1]], add=True)

            @pl.when(r < _G // _NBUF - 1)
            def _():
                pltpu.sync_copy(idx2_hbm.at[base + g + _NBUF], ibufs[b])
                pltpu.async_copy(xs_hbm.at[ibufs[b].at[0]], bufs[b], gsems[b])

        return carry

    lax.fori_loop(0, _G // _NBUF, round_body, 0)
    plsc.subcore_barrier()
    pltpu.sync_copy(acc_sh.at[pl.ds(s * _RPT, _RPT)],
                    out_hbm.at[c, pl.ds(s * _RPT, _RPT)])


_RB = 2048


def _scale_body(x_ref, d_ref, o_ref):
    o_ref[...] = x_ref[...] * d_ref[...]


def _layer_body(p_ref, din_ref, dout_ref, w1_ref, b1_ref, w2_ref, o_ref):
    t = (p_ref[0] + p_ref[1]) * din_ref[...]
    u = jnp.dot(t, w1_ref[...], preferred_element_type=jnp.float32) + b1_ref[0:1, :]
    h = jnp.maximum(u, 0.0)
    o_ref[...] = jnp.dot(h, w2_ref[...], preferred_element_type=jnp.float32) * dout_ref[...]


def _final_body(p_ref, din_ref, b2_ref, o_ref):
    o_ref[...] = (p_ref[0] + p_ref[1]) * din_ref[...] + b2_ref[0:1, :]


def _tc_scale(x, d):
    return pl.pallas_call(
        _scale_body,
        grid=(_NP // _RB,),
        in_specs=[
            pl.BlockSpec((_RB, 128), lambda i: (i, 0)),
            pl.BlockSpec((_RB, 128), lambda i: (i, 0)),
        ],
        out_specs=pl.BlockSpec((_RB, 128), lambda i: (i, 0)),
        out_shape=jax.ShapeDtypeStruct((_NP, 128), jnp.float32),
    )(x, d)


def _tc_layer(parts, din, dout, W1, b1b, W2):
    return pl.pallas_call(
        _layer_body,
        grid=(_NP // _RB,),
        in_specs=[
            pl.BlockSpec((2, _RB, 128), lambda i: (0, i, 0)),
            pl.BlockSpec((_RB, 128), lambda i: (i, 0)),
            pl.BlockSpec((_RB, 128), lambda i: (i, 0)),
            pl.BlockSpec((_IN, _HID), lambda i: (0, 0)),
            pl.BlockSpec((8, _HID), lambda i: (0, 0)),
            pl.BlockSpec((_HID, _OUT), lambda i: (0, 0)),
        ],
        out_specs=pl.BlockSpec((_RB, 128), lambda i: (i, 0)),
        out_shape=jax.ShapeDtypeStruct((_NP, 128), jnp.float32),
    )(parts, din, dout, W1, b1b, W2)


def _tc_final(parts, din, b2b):
    return pl.pallas_call(
        _final_body,
        grid=(_NP // _RB,),
        in_specs=[
            pl.BlockSpec((2, _RB, 128), lambda i: (0, i, 0)),
            pl.BlockSpec((_RB, 128), lambda i: (i, 0)),
            pl.BlockSpec((8, 128), lambda i: (0, 0)),
        ],
        out_specs=pl.BlockSpec((_RB, 128), lambda i: (i, 0)),
        out_shape=jax.ShapeDtypeStruct((_NP, 128), jnp.float32),
    )(parts, din, b2b)


def kernel(features, edge_index, W1, b1, W2, b2):
    src = edge_index[0]
    dst = edge_index[1]
    pad = _EPAD - _E
    pad_idx = _N + (jnp.arange(pad, dtype=jnp.int32) % (_NP - _N))
    src_p = jnp.concatenate([src, pad_idx])
    dst_p = jnp.concatenate([dst, pad_idx])
    idx2 = jnp.stack([src_p.reshape(_EPAD // _GRP, _GRP),
                      dst_p.reshape(_EPAD // _GRP, _GRP)], axis=1)
    feat_p = jnp.pad(features, ((0, _NP - _N), (0, 0)))

    degs, degd = _sc_degrees(src_p, dst_p)
    deg_out = jnp.sum(degs, axis=0)
    deg_in = jnp.sum(degd, axis=0)
    d_out = lax.rsqrt(jnp.clip(deg_out, 1.0))
    d_in = lax.rsqrt(jnp.clip(deg_in, 1.0))
    d_out_b = jnp.broadcast_to(d_out[:, None], (_NP, 128))
    d_in_b = jnp.broadcast_to(d_in[:, None], (_NP, 128))
    b1b = jnp.broadcast_to(b1[None, :], (8, _HID))
    b2b = jnp.broadcast_to(b2[None, :], (8, _OUT))

    xs = _tc_scale(feat_p, d_out_b)
    parts1 = _sc_aggregate(xs, idx2)
    z = _tc_layer(parts1, d_in_b, d_out_b, W1, b1b, W2)
    parts2 = _sc_aggregate(z, idx2)
    out_full = _tc_final(parts2, d_in_b, b2b)
    return out_full[:_N]
